# baseline (speedup 1.0000x reference)
import functools

import jax
import jax.numpy as jnp
from jax import lax
from jax.experimental import pallas as pl
from jax.experimental.pallas import tpu as pltpu
from jax.experimental.pallas import tpu_sc as plsc

B, C, N, K = 16, 96, 1024, 16
C2 = 2 * C
COUT = 96
NSAMP = B * N
GN = 128
NSUB = GN // 16


NGROUPS = N // GN
KH = K // 2
CP = C // 2


def _sc_body(xp_hbm, e0_hbm, e1_hbm, msg_hbm, table, idx0, idx1, msgbuf):
    b = lax.axis_index("s")
    lanes = lax.iota(jnp.int32, 16)
    ninf = jnp.full((32,), -jnp.inf, jnp.bfloat16)

    pltpu.sync_copy(xp_hbm.at[b], table)

    def group(g, carry):
        n0 = g * GN
        pltpu.sync_copy(e0_hbm.at[b, pl.ds(n0 * K, GN * K)], idx0)
        pltpu.sync_copy(e1_hbm.at[b, pl.ds(n0 * K, GN * K)], idx1)

        def sub(sg, sc_):
            nodes = sg * 16 + lanes

            def khalf(k0, first):
                rs = [plsc.load_gather(idx0, [nodes * K + (k0 + k)])
                      for k in range(KH)]
                rd = [plsc.load_gather(idx1, [nodes * K + (k0 + k)])
                      for k in range(KH)]

                def cbody(c, cc):
                    cvec = jnp.full((16,), c, jnp.int32)
                    acc = ninf
                    for k in range(KH):
                        s = plsc.load_gather(table, [cvec, rs[k]])
                        d = plsc.load_gather(table, [cvec, rd[k]])
                        diff = (plsc.bitcast(s, jnp.bfloat16)
                                - plsc.bitcast(d, jnp.bfloat16))
                        acc = jnp.maximum(acc, diff)
                    if not first:
                        prev = plsc.load_gather(msgbuf, [cvec, nodes])
                        acc = jnp.maximum(acc, plsc.bitcast(prev, jnp.bfloat16))
                    plsc.store_scatter(msgbuf, [cvec, nodes],
                                       plsc.bitcast(acc, jnp.int32))
                    return cc

                lax.fori_loop(0, CP, cbody, 0)

            khalf(0, True)
            khalf(KH, False)
            return sc_

        lax.fori_loop(0, NSUB, sub, 0)
        pltpu.sync_copy(msgbuf, msg_hbm.at[b, :, pl.ds(n0, GN)])
        return carry

    lax.fori_loop(0, NGROUPS, group, 0)


@functools.cache
def _sc_msg():
    return pl.kernel(
        _sc_body,
        out_type=jax.ShapeDtypeStruct((B, CP, N), jnp.int32),
        mesh=plsc.VectorSubcoreMesh(core_axis_name="c", subcore_axis_name="s",
                                    num_cores=1, num_subcores=16),
        compiler_params=pltpu.CompilerParams(needs_layout_passes=False),
        scratch_types=[
            pltpu.VMEM((CP, N), jnp.int32),
            pltpu.VMEM((GN * K,), jnp.int32),
            pltpu.VMEM((GN * K,), jnp.int32),
            pltpu.VMEM((CP, GN), jnp.int32),
        ],
    )


_DOT = dict(preferred_element_type=jnp.float32, precision=lax.Precision.HIGHEST)


def _gelu(x):
    return 0.5 * x * (1.0 + lax.erf(x * 0.7071067811865476))


def _h1(x_ref, m_ref, wx_ref, wme_ref, wmo_ref, b1_ref):
    m = m_ref[0]
    mlo = lax.bitcast_convert_type(m << 16, jnp.float32)
    mhi = lax.bitcast_convert_type(m & jnp.int32(-65536), jnp.float32)
    return (jnp.dot(wx_ref[...], x_ref[0], **_DOT)
            + jnp.dot(wme_ref[...], mlo, **_DOT)
            + jnp.dot(wmo_ref[...], mhi, **_DOT) + b1_ref[...])


def _p1(x_ref, m_ref, wx_ref, wme_ref, wmo_ref, b1_ref, s_ref):
    h = _h1(x_ref, m_ref, wx_ref, wme_ref, wmo_ref, b1_ref)

    @pl.when(pl.program_id(0) == 0)
    def _():
        s_ref[...] = jnp.zeros_like(s_ref)

    s_ref[...] += jnp.concatenate(
        [jnp.sum(h, 1, keepdims=True), jnp.sum(h * h, 1, keepdims=True)], 1)


def _norm_gelu(h, s_ref, g_ref, be_ref):
    inv = 1.0 / NSAMP
    mean = s_ref[:, 0:1] * inv
    var = s_ref[:, 1:2] * inv - mean * mean
    scale = g_ref[...] * lax.rsqrt(var + 1e-5)
    return _gelu((h - mean) * scale + be_ref[...])


def _p2(x_ref, m_ref, wx_ref, wme_ref, wmo_ref, b1_ref,
        s1_ref, g1_ref, be1_ref, w2_ref, b2_ref, h2_ref, s2_ref):
    h = _h1(x_ref, m_ref, wx_ref, wme_ref, wmo_ref, b1_ref)
    g = _norm_gelu(h, s1_ref, g1_ref, be1_ref)
    h2 = jnp.dot(w2_ref[...], g, **_DOT) + b2_ref[...]
    h2_ref[0] = h2

    @pl.when(pl.program_id(0) == 0)
    def _():
        s2_ref[...] = jnp.zeros_like(s2_ref)

    s2_ref[...] += jnp.concatenate(
        [jnp.sum(h2, 1, keepdims=True), jnp.sum(h2 * h2, 1, keepdims=True)], 1)


def _p3(h2_ref, s2_ref, g2_ref, be2_ref, o_ref):
    o_ref[0] = _norm_gelu(h2_ref[0], s2_ref, g2_ref, be2_ref)


def _full(shape):
    return pl.BlockSpec(shape, lambda b: tuple(0 for _ in shape))


def kernel(x, edge_idx, conv1_w, conv1_b, bn1_g, bn1_b,
           conv2_w, conv2_b, bn2_g, bn2_b):
    xc = x.reshape(B, C, N)
    e0 = edge_idx[0].reshape(B, N * K)
    e1 = edge_idx[1].reshape(B, N * K)

    u = lax.bitcast_convert_type(xc.astype(jnp.bfloat16),
                                 jnp.uint16).astype(jnp.uint32)
    xp = lax.bitcast_convert_type((u[:, 1::2] << 16) | u[:, 0::2], jnp.int32)

    msg = _sc_msg()(xp, e0, e1)

    w1x = conv1_w[:, 0::2]
    w1m = conv1_w[:, 1::2]
    w1me = w1m[:, 0::2]
    w1mo = w1m[:, 1::2]
    b1 = conv1_b.reshape(C2, 1)
    g1 = bn1_g.reshape(C2, 1)
    be1 = bn1_b.reshape(C2, 1)
    b2 = conv2_b.reshape(COUT, 1)
    g2 = bn2_g.reshape(COUT, 1)
    be2 = bn2_b.reshape(COUT, 1)

    xspec = pl.BlockSpec((1, C, N), lambda b: (b, 0, 0))
    mspec = pl.BlockSpec((1, CP, N), lambda b: (b, 0, 0))
    sspec1 = _full((C2, 2))
    sspec2 = _full((COUT, 2))

    s1 = pl.pallas_call(
        _p1,
        grid=(B,),
        in_specs=[xspec, mspec, _full((C2, C)), _full((C2, CP)),
                  _full((C2, CP)), _full((C2, 1))],
        out_specs=sspec1,
        out_shape=jax.ShapeDtypeStruct((C2, 2), jnp.float32),
    )(xc, msg, w1x, w1me, w1mo, b1)

    h2, s2 = pl.pallas_call(
        _p2,
        grid=(B,),
        in_specs=[xspec, mspec, _full((C2, C)), _full((C2, CP)),
                  _full((C2, CP)), _full((C2, 1)),
                  sspec1, _full((C2, 1)), _full((C2, 1)),
                  _full((COUT, C2)), _full((COUT, 1))],
        out_specs=[pl.BlockSpec((1, COUT, N), lambda b: (b, 0, 0)), sspec2],
        out_shape=[jax.ShapeDtypeStruct((B, COUT, N), jnp.float32),
                   jax.ShapeDtypeStruct((COUT, 2), jnp.float32)],
    )(xc, msg, w1x, w1me, w1mo, b1, s1, g1, be1, conv2_w, b2)

    out = pl.pallas_call(
        _p3,
        grid=(B,),
        in_specs=[pl.BlockSpec((1, COUT, N), lambda b: (b, 0, 0)),
                  sspec2, _full((COUT, 1)), _full((COUT, 1))],
        out_specs=pl.BlockSpec((1, COUT, N), lambda b: (b, 0, 0)),
        out_shape=jax.ShapeDtypeStruct((B, COUT, N), jnp.float32),
    )(h2, s2, g2, be2)

    return out.reshape(B, COUT, 32, 32)


# speedup vs baseline: 1.2708x; 1.2708x over previous
import functools

import jax
import jax.numpy as jnp
from jax import lax
from jax.experimental import pallas as pl
from jax.experimental.pallas import tpu as pltpu
from jax.experimental.pallas import tpu_sc as plsc

B, C, N, K = 16, 96, 1024, 16
C2 = 2 * C
COUT = 96
NSAMP = B * N
GN = 128
NSUB = GN // 16


NODES_PER_W = (B * N) // 32
NGROUPS = NODES_PER_W // GN
KH = K // 2
CP = C // 2


def _sc_body(xp_hbm, e0_hbm, e1_hbm, msg_hbm, table,
             idx0a, idx1a, idx0b, idx1b, msgbufa, msgbufb,
             stab, sina, sinb, souta, soutb):
    cid = lax.axis_index("c")
    sid = lax.axis_index("s")
    wid = sid * 2 + cid
    b = wid // 2
    half = wid % 2
    lanes = lax.iota(jnp.int32, 16)
    ninf = jnp.full((32,), -jnp.inf, jnp.bfloat16)

    idxs = [(idx0a, idx1a), (idx0b, idx1b)]
    mbufs = [msgbufa, msgbufb]
    sins = [sina, sinb]
    souts = [souta, soutb]

    def idx_start(g, slot):
        n0 = half * NODES_PER_W + g * GN
        i0, i1 = idxs[slot]
        h0 = pltpu.async_copy(e0_hbm.at[b, pl.ds(n0 * K, GN * K)], i0,
                              sins[slot])
        h1 = pltpu.async_copy(e1_hbm.at[b, pl.ds(n0 * K, GN * K)], i1,
                              sins[slot])
        return (h0, h1)

    htab = pltpu.async_copy(xp_hbm.at[b], table, stab)
    pend = idx_start(0, 0)
    htab.wait()

    out_handles = [None, None]
    for g in range(NGROUPS):
        slot = g % 2
        for h in pend:
            h.wait()
        if g + 1 < NGROUPS:
            pend = idx_start(g + 1, (g + 1) % 2)
        if out_handles[slot] is not None:
            out_handles[slot].wait()
        idx0, idx1 = idxs[slot]
        msgbuf = mbufs[slot]

        def sub(sg, sc_):
            nodes = sg * 16 + lanes

            def khalf(k0, first):
                rs = [plsc.load_gather(idx0, [nodes * K + (k0 + k)])
                      for k in range(KH)]
                rd = [plsc.load_gather(idx1, [nodes * K + (k0 + k)])
                      for k in range(KH)]

                def cbody(c, cc):
                    cvec = jnp.full((16,), c, jnp.int32)
                    acc = ninf
                    for k in range(KH):
                        s = plsc.load_gather(table, [cvec, rs[k]])
                        d = plsc.load_gather(table, [cvec, rd[k]])
                        diff = (plsc.bitcast(s, jnp.bfloat16)
                                - plsc.bitcast(d, jnp.bfloat16))
                        acc = jnp.maximum(acc, diff)
                    if not first:
                        prev = plsc.load_gather(msgbuf, [cvec, nodes])
                        acc = jnp.maximum(acc, plsc.bitcast(prev, jnp.bfloat16))
                    plsc.store_scatter(msgbuf, [cvec, nodes],
                                       plsc.bitcast(acc, jnp.int32))
                    return cc

                lax.fori_loop(0, CP, cbody, 0)

            khalf(0, True)
            khalf(KH, False)
            return sc_

        lax.fori_loop(0, NSUB, sub, 0)
        n0 = half * NODES_PER_W + g * GN
        out_handles[slot] = pltpu.async_copy(
            msgbuf, msg_hbm.at[b, :, pl.ds(n0, GN)], souts[slot])

    for h in out_handles:
        if h is not None:
            h.wait()


@functools.cache
def _sc_msg():
    return pl.kernel(
        _sc_body,
        out_type=jax.ShapeDtypeStruct((B, CP, N), jnp.int32),
        mesh=plsc.VectorSubcoreMesh(core_axis_name="c", subcore_axis_name="s",
                                    num_cores=2, num_subcores=16),
        compiler_params=pltpu.CompilerParams(needs_layout_passes=False),
        scratch_types=[
            pltpu.VMEM((CP, N), jnp.int32),
            pltpu.VMEM((GN * K,), jnp.int32),
            pltpu.VMEM((GN * K,), jnp.int32),
            pltpu.VMEM((GN * K,), jnp.int32),
            pltpu.VMEM((GN * K,), jnp.int32),
            pltpu.VMEM((CP, GN), jnp.int32),
            pltpu.VMEM((CP, GN), jnp.int32),
            pltpu.SemaphoreType.DMA,
            pltpu.SemaphoreType.DMA,
            pltpu.SemaphoreType.DMA,
            pltpu.SemaphoreType.DMA,
            pltpu.SemaphoreType.DMA,
        ],
    )


_DOT = dict(preferred_element_type=jnp.float32, precision=lax.Precision.HIGHEST)


def _gelu(x):
    return 0.5 * x * (1.0 + lax.erf(x * 0.7071067811865476))


def _h1(x_ref, m_ref, wx_ref, wme_ref, wmo_ref, b1_ref):
    m = m_ref[0]
    mlo = lax.bitcast_convert_type(m << 16, jnp.float32)
    mhi = lax.bitcast_convert_type(m & jnp.int32(-65536), jnp.float32)
    return (jnp.dot(wx_ref[...], x_ref[0], **_DOT)
            + jnp.dot(wme_ref[...], mlo, **_DOT)
            + jnp.dot(wmo_ref[...], mhi, **_DOT) + b1_ref[...])


def _p1(x_ref, m_ref, wx_ref, wme_ref, wmo_ref, b1_ref, s_ref):
    h = _h1(x_ref, m_ref, wx_ref, wme_ref, wmo_ref, b1_ref)

    @pl.when(pl.program_id(0) == 0)
    def _():
        s_ref[...] = jnp.zeros_like(s_ref)

    s_ref[...] += jnp.concatenate(
        [jnp.sum(h, 1, keepdims=True), jnp.sum(h * h, 1, keepdims=True)], 1)


def _norm_gelu(h, s_ref, g_ref, be_ref):
    inv = 1.0 / NSAMP
    mean = s_ref[:, 0:1] * inv
    var = s_ref[:, 1:2] * inv - mean * mean
    scale = g_ref[...] * lax.rsqrt(var + 1e-5)
    return _gelu((h - mean) * scale + be_ref[...])


def _p2(x_ref, m_ref, wx_ref, wme_ref, wmo_ref, b1_ref,
        s1_ref, g1_ref, be1_ref, w2_ref, b2_ref, h2_ref, s2_ref):
    h = _h1(x_ref, m_ref, wx_ref, wme_ref, wmo_ref, b1_ref)
    g = _norm_gelu(h, s1_ref, g1_ref, be1_ref)
    h2 = jnp.dot(w2_ref[...], g, **_DOT) + b2_ref[...]
    h2_ref[0] = h2

    @pl.when(pl.program_id(0) == 0)
    def _():
        s2_ref[...] = jnp.zeros_like(s2_ref)

    s2_ref[...] += jnp.concatenate(
        [jnp.sum(h2, 1, keepdims=True), jnp.sum(h2 * h2, 1, keepdims=True)], 1)


def _p3(h2_ref, s2_ref, g2_ref, be2_ref, o_ref):
    o_ref[0] = _norm_gelu(h2_ref[0], s2_ref, g2_ref, be2_ref)


def _full(shape):
    return pl.BlockSpec(shape, lambda b: tuple(0 for _ in shape))


def kernel(x, edge_idx, conv1_w, conv1_b, bn1_g, bn1_b,
           conv2_w, conv2_b, bn2_g, bn2_b):
    xc = x.reshape(B, C, N)
    e0 = edge_idx[0].reshape(B, N * K)
    e1 = edge_idx[1].reshape(B, N * K)

    u = lax.bitcast_convert_type(xc.astype(jnp.bfloat16),
                                 jnp.uint16).astype(jnp.uint32)
    xp = lax.bitcast_convert_type((u[:, 1::2] << 16) | u[:, 0::2], jnp.int32)

    msg = _sc_msg()(xp, e0, e1)

    w1x = conv1_w[:, 0::2]
    w1m = conv1_w[:, 1::2]
    w1me = w1m[:, 0::2]
    w1mo = w1m[:, 1::2]
    b1 = conv1_b.reshape(C2, 1)
    g1 = bn1_g.reshape(C2, 1)
    be1 = bn1_b.reshape(C2, 1)
    b2 = conv2_b.reshape(COUT, 1)
    g2 = bn2_g.reshape(COUT, 1)
    be2 = bn2_b.reshape(COUT, 1)

    xspec = pl.BlockSpec((1, C, N), lambda b: (b, 0, 0))
    mspec = pl.BlockSpec((1, CP, N), lambda b: (b, 0, 0))
    sspec1 = _full((C2, 2))
    sspec2 = _full((COUT, 2))

    s1 = pl.pallas_call(
        _p1,
        grid=(B,),
        in_specs=[xspec, mspec, _full((C2, C)), _full((C2, CP)),
                  _full((C2, CP)), _full((C2, 1))],
        out_specs=sspec1,
        out_shape=jax.ShapeDtypeStruct((C2, 2), jnp.float32),
    )(xc, msg, w1x, w1me, w1mo, b1)

    h2, s2 = pl.pallas_call(
        _p2,
        grid=(B,),
        in_specs=[xspec, mspec, _full((C2, C)), _full((C2, CP)),
                  _full((C2, CP)), _full((C2, 1)),
                  sspec1, _full((C2, 1)), _full((C2, 1)),
                  _full((COUT, C2)), _full((COUT, 1))],
        out_specs=[pl.BlockSpec((1, COUT, N), lambda b: (b, 0, 0)), sspec2],
        out_shape=[jax.ShapeDtypeStruct((B, COUT, N), jnp.float32),
                   jax.ShapeDtypeStruct((COUT, 2), jnp.float32)],
    )(xc, msg, w1x, w1me, w1mo, b1, s1, g1, be1, conv2_w, b2)

    out = pl.pallas_call(
        _p3,
        grid=(B,),
        in_specs=[pl.BlockSpec((1, COUT, N), lambda b: (b, 0, 0)),
                  sspec2, _full((COUT, 1)), _full((COUT, 1))],
        out_specs=pl.BlockSpec((1, COUT, N), lambda b: (b, 0, 0)),
        out_shape=jax.ShapeDtypeStruct((B, COUT, N), jnp.float32),
    )(h2, s2, g2, be2)

    return out.reshape(B, COUT, 32, 32)


# speedup vs baseline: 1.2767x; 1.0046x over previous
import functools

import jax
import jax.numpy as jnp
from jax import lax
from jax.experimental import pallas as pl
from jax.experimental.pallas import tpu as pltpu
from jax.experimental.pallas import tpu_sc as plsc

B, C, N, K = 16, 96, 1024, 16
C2 = 2 * C
COUT = 96
NSAMP = B * N
GN = 128
NSUB = GN // 16


NODES_PER_W = (B * N) // 32
NGROUPS = NODES_PER_W // GN
KH = K // 2
CP = C // 2


def _sc_body(xp_hbm, e0_hbm, e1_hbm, msg_hbm, table,
             idx0a, idx1a, idx0b, idx1b, msgbufa, msgbufb,
             stab, sina, sinb, souta, soutb):
    cid = lax.axis_index("c")
    sid = lax.axis_index("s")
    wid = sid * 2 + cid
    b = wid // 2
    half = wid % 2
    lanes = lax.iota(jnp.int32, 16)
    ninf = jnp.full((32,), -jnp.inf, jnp.bfloat16)

    idxs = [(idx0a, idx1a), (idx0b, idx1b)]
    mbufs = [msgbufa, msgbufb]
    sins = [sina, sinb]
    souts = [souta, soutb]

    def idx_start(g, slot):
        n0 = half * NODES_PER_W + g * GN
        i0, i1 = idxs[slot]
        h0 = pltpu.async_copy(e0_hbm.at[b, pl.ds(n0 * K, GN * K)], i0,
                              sins[slot])
        h1 = pltpu.async_copy(e1_hbm.at[b, pl.ds(n0 * K, GN * K)], i1,
                              sins[slot])
        return (h0, h1)

    htab = pltpu.async_copy(xp_hbm.at[b], table, stab)
    pend = idx_start(0, 0)
    htab.wait()

    out_handles = [None, None]
    for g in range(NGROUPS):
        slot = g % 2
        for h in pend:
            h.wait()
        if g + 1 < NGROUPS:
            pend = idx_start(g + 1, (g + 1) % 2)
        if out_handles[slot] is not None:
            out_handles[slot].wait()
        idx0, idx1 = idxs[slot]
        msgbuf = mbufs[slot]

        def sub(sg, sc_):
            nodes = sg * 16 + lanes

            def khalf(k0, first):
                rs = [plsc.load_gather(idx0, [nodes * K + (k0 + k)])
                      for k in range(KH)]
                rd = [plsc.load_gather(idx1, [nodes * K + (k0 + k)])
                      for k in range(KH)]

                def cbody(c, cc):
                    cvec = jnp.full((16,), c, jnp.int32)
                    acc = ninf
                    for k in range(KH):
                        s = plsc.load_gather(table, [cvec, rs[k]])
                        d = plsc.load_gather(table, [cvec, rd[k]])
                        diff = (plsc.bitcast(s, jnp.bfloat16)
                                - plsc.bitcast(d, jnp.bfloat16))
                        acc = jnp.maximum(acc, diff)
                    if not first:
                        prev = plsc.load_gather(msgbuf, [cvec, nodes])
                        acc = jnp.maximum(acc, plsc.bitcast(prev, jnp.bfloat16))
                    plsc.store_scatter(msgbuf, [cvec, nodes],
                                       plsc.bitcast(acc, jnp.int32))
                    return cc

                lax.fori_loop(0, CP, cbody, 0)

            khalf(0, True)
            khalf(KH, False)
            return sc_

        lax.fori_loop(0, NSUB, sub, 0)
        n0 = half * NODES_PER_W + g * GN
        out_handles[slot] = pltpu.async_copy(
            msgbuf, msg_hbm.at[b, :, pl.ds(n0, GN)], souts[slot])

    for h in out_handles:
        if h is not None:
            h.wait()


@functools.cache
def _sc_msg():
    return pl.kernel(
        _sc_body,
        out_type=jax.ShapeDtypeStruct((B, CP, N), jnp.int32),
        mesh=plsc.VectorSubcoreMesh(core_axis_name="c", subcore_axis_name="s",
                                    num_cores=2, num_subcores=16),
        compiler_params=pltpu.CompilerParams(needs_layout_passes=False),
        scratch_types=[
            pltpu.VMEM((CP, N), jnp.int32),
            pltpu.VMEM((GN * K,), jnp.int32),
            pltpu.VMEM((GN * K,), jnp.int32),
            pltpu.VMEM((GN * K,), jnp.int32),
            pltpu.VMEM((GN * K,), jnp.int32),
            pltpu.VMEM((CP, GN), jnp.int32),
            pltpu.VMEM((CP, GN), jnp.int32),
            pltpu.SemaphoreType.DMA,
            pltpu.SemaphoreType.DMA,
            pltpu.SemaphoreType.DMA,
            pltpu.SemaphoreType.DMA,
            pltpu.SemaphoreType.DMA,
        ],
    )


_DOT = dict(preferred_element_type=jnp.float32, precision=lax.Precision.HIGHEST)


def _gelu(x):
    return 0.5 * x * (1.0 + lax.erf(x * 0.7071067811865476))


def _h1(x_ref, m_ref, wx_ref, wme_ref, wmo_ref, b1_ref):
    m = m_ref[0]
    mlo = lax.bitcast_convert_type(m << 16, jnp.float32)
    mhi = lax.bitcast_convert_type(m & jnp.int32(-65536), jnp.float32)
    return (jnp.dot(wx_ref[...], x_ref[0], **_DOT)
            + jnp.dot(wme_ref[...], mlo, **_DOT)
            + jnp.dot(wmo_ref[...], mhi, **_DOT) + b1_ref[...])


def _norm_gelu(h, s_ref, g_ref, be_ref):
    inv = 1.0 / NSAMP
    mean = s_ref[:, 0:1] * inv
    var = s_ref[:, 1:2] * inv - mean * mean
    scale = g_ref[...] * lax.rsqrt(var + 1e-5)
    return _gelu((h - mean) * scale + be_ref[...])


def _stats(h, s_ref, is_first):
    @pl.when(is_first)
    def _():
        s_ref[...] = jnp.zeros_like(s_ref)

    s_ref[...] += jnp.concatenate(
        [jnp.sum(h, 1, keepdims=True), jnp.sum(h * h, 1, keepdims=True)], 1)


def _mlp(x_ref, m_ref, wx_ref, wme_ref, wmo_ref, b1_ref, g1_ref, be1_ref,
         w2_ref, b2_ref, g2_ref, be2_ref, o_ref, s1_ref, s2_ref, h2s_ref):
    p = pl.program_id(0)
    b = pl.program_id(1)

    @pl.when(p == 0)
    def _():
        _stats(_h1(x_ref, m_ref, wx_ref, wme_ref, wmo_ref, b1_ref),
               s1_ref, b == 0)

    @pl.when(p == 1)
    def _():
        h = _h1(x_ref, m_ref, wx_ref, wme_ref, wmo_ref, b1_ref)
        g = _norm_gelu(h, s1_ref, g1_ref, be1_ref)
        h2 = jnp.dot(w2_ref[...], g, **_DOT) + b2_ref[...]
        h2s_ref[b] = h2
        _stats(h2, s2_ref, b == 0)

    @pl.when(p == 2)
    def _():
        o_ref[0] = _norm_gelu(h2s_ref[b], s2_ref, g2_ref, be2_ref)


def _full(shape):
    return pl.BlockSpec(shape, lambda b: tuple(0 for _ in shape))


def kernel(x, edge_idx, conv1_w, conv1_b, bn1_g, bn1_b,
           conv2_w, conv2_b, bn2_g, bn2_b):
    xc = x.reshape(B, C, N)
    e0 = edge_idx[0].reshape(B, N * K)
    e1 = edge_idx[1].reshape(B, N * K)

    u = lax.bitcast_convert_type(xc.astype(jnp.bfloat16),
                                 jnp.uint16).astype(jnp.uint32)
    xp = lax.bitcast_convert_type((u[:, 1::2] << 16) | u[:, 0::2], jnp.int32)

    msg = _sc_msg()(xp, e0, e1)

    w1x = conv1_w[:, 0::2]
    w1m = conv1_w[:, 1::2]
    w1me = w1m[:, 0::2]
    w1mo = w1m[:, 1::2]
    b1 = conv1_b.reshape(C2, 1)
    g1 = bn1_g.reshape(C2, 1)
    be1 = bn1_b.reshape(C2, 1)
    b2 = conv2_b.reshape(COUT, 1)
    g2 = bn2_g.reshape(COUT, 1)
    be2 = bn2_b.reshape(COUT, 1)

    xspec = pl.BlockSpec((1, C, N), lambda p, b: (b, 0, 0))
    mspec = pl.BlockSpec((1, CP, N), lambda p, b: (b, 0, 0))

    def _fullpb(shape):
        return pl.BlockSpec(shape, lambda p, b: tuple(0 for _ in shape))

    out = pl.pallas_call(
        _mlp,
        grid=(3, B),
        in_specs=[xspec, mspec, _fullpb((C2, C)), _fullpb((C2, CP)),
                  _fullpb((C2, CP)), _fullpb((C2, 1)), _fullpb((C2, 1)),
                  _fullpb((C2, 1)), _fullpb((COUT, C2)), _fullpb((COUT, 1)),
                  _fullpb((COUT, 1)), _fullpb((COUT, 1))],
        out_specs=pl.BlockSpec((1, COUT, N), lambda p, b: (b, 0, 0)),
        out_shape=jax.ShapeDtypeStruct((B, COUT, N), jnp.float32),
        scratch_shapes=[
            pltpu.VMEM((C2, 2), jnp.float32),
            pltpu.VMEM((COUT, 2), jnp.float32),
            pltpu.VMEM((B, COUT, N), jnp.float32),
        ],
    )(xc, msg, w1x, w1me, w1mo, b1, g1, be1, conv2_w, b2, g2, be2)

    return out.reshape(B, COUT, 32, 32)


# speedup vs baseline: 1.4007x; 1.0972x over previous
import functools

import jax
import jax.numpy as jnp
from jax import lax
from jax.experimental import pallas as pl
from jax.experimental.pallas import tpu as pltpu
from jax.experimental.pallas import tpu_sc as plsc

B, C, N, K = 16, 96, 1024, 16
C2 = 2 * C
COUT = 96
NSAMP = B * N
GN = 128
NSUB = GN // 16


NODES_PER_W = (B * N) // 32
NGROUPS = NODES_PER_W // GN
KH = K // 2
CP = C // 2


CCH = 32


def _sc_body(xc_hbm, e0_hbm, e1_hbm, msg_hbm, chunk, table,
             idx0a, idx1a, idx0b, idx1b, msgbufa, msgbufb,
             stab, sina, sinb, souta, soutb):
    cid = lax.axis_index("c")
    sid = lax.axis_index("s")
    wid = sid * 2 + cid
    b = wid // 2
    half = wid % 2
    lanes = lax.iota(jnp.int32, 16)
    ninf = jnp.full((32,), -jnp.inf, jnp.bfloat16)

    idxs = [(idx0a, idx1a), (idx0b, idx1b)]
    mbufs = [msgbufa, msgbufb]
    sins = [sina, sinb]
    souts = [souta, soutb]

    def idx_start(g, slot):
        n0 = half * NODES_PER_W + g * GN
        i0, i1 = idxs[slot]
        h0 = pltpu.async_copy(e0_hbm.at[b, pl.ds(n0 * K, GN * K)], i0,
                              sins[slot])
        h1 = pltpu.async_copy(e1_hbm.at[b, pl.ds(n0 * K, GN * K)], i1,
                              sins[slot])
        return (h0, h1)

    pend = idx_start(0, 0)

    for ci in range(C // CCH):
        pltpu.async_copy(xc_hbm.at[b, pl.ds(ci * CCH, CCH)], chunk,
                         stab).wait()

        def packv(v, pc_):
            for pp in range(CCH // 2):
                a = chunk[2 * pp, pl.ds(v * 16, 16)]
                o = chunk[2 * pp + 1, pl.ds(v * 16, 16)]
                w = plsc.bitcast(
                    plsc.pack(a, o, format=plsc.PackFormat.INTERLEAVED),
                    jnp.int32)
                table[ci * (CCH // 2) + pp, pl.ds(v * 16, 16)] = w
            return pc_

        lax.fori_loop(0, N // 16, packv, 0)

    out_handles = [None, None]
    for g in range(NGROUPS):
        slot = g % 2
        for h in pend:
            h.wait()
        if g + 1 < NGROUPS:
            pend = idx_start(g + 1, (g + 1) % 2)
        if out_handles[slot] is not None:
            out_handles[slot].wait()
        idx0, idx1 = idxs[slot]
        msgbuf = mbufs[slot]

        def sub(sg, sc_):
            nodes = sg * 16 + lanes

            def khalf(k0, first):
                rs = [plsc.load_gather(idx0, [nodes * K + (k0 + k)])
                      for k in range(KH)]
                rd = [plsc.load_gather(idx1, [nodes * K + (k0 + k)])
                      for k in range(KH)]

                def cbody(c, cc):
                    cvec = jnp.full((16,), c, jnp.int32)
                    acc = ninf
                    for k in range(KH):
                        s = plsc.load_gather(table, [cvec, rs[k]])
                        d = plsc.load_gather(table, [cvec, rd[k]])
                        diff = (plsc.bitcast(s, jnp.bfloat16)
                                - plsc.bitcast(d, jnp.bfloat16))
                        acc = jnp.maximum(acc, diff)
                    if not first:
                        prev = plsc.load_gather(msgbuf, [cvec, nodes])
                        acc = jnp.maximum(acc, plsc.bitcast(prev, jnp.bfloat16))
                    plsc.store_scatter(msgbuf, [cvec, nodes],
                                       plsc.bitcast(acc, jnp.int32))
                    return cc

                lax.fori_loop(0, CP, cbody, 0)

            khalf(0, True)
            khalf(KH, False)
            return sc_

        lax.fori_loop(0, NSUB, sub, 0)
        n0 = half * NODES_PER_W + g * GN
        out_handles[slot] = pltpu.async_copy(
            msgbuf, msg_hbm.at[b, :, pl.ds(n0, GN)], souts[slot])

    for h in out_handles:
        if h is not None:
            h.wait()


@functools.cache
def _sc_msg():
    return pl.kernel(
        _sc_body,
        out_type=jax.ShapeDtypeStruct((B, CP, N), jnp.int32),
        mesh=plsc.VectorSubcoreMesh(core_axis_name="c", subcore_axis_name="s",
                                    num_cores=2, num_subcores=16),
        compiler_params=pltpu.CompilerParams(needs_layout_passes=False),
        scratch_types=[
            pltpu.VMEM((CCH, N), jnp.float32),
            pltpu.VMEM((CP, N), jnp.int32),
            pltpu.VMEM((GN * K,), jnp.int32),
            pltpu.VMEM((GN * K,), jnp.int32),
            pltpu.VMEM((GN * K,), jnp.int32),
            pltpu.VMEM((GN * K,), jnp.int32),
            pltpu.VMEM((CP, GN), jnp.int32),
            pltpu.VMEM((CP, GN), jnp.int32),
            pltpu.SemaphoreType.DMA,
            pltpu.SemaphoreType.DMA,
            pltpu.SemaphoreType.DMA,
            pltpu.SemaphoreType.DMA,
            pltpu.SemaphoreType.DMA,
        ],
    )


_DOT = dict(preferred_element_type=jnp.float32, precision=lax.Precision.HIGHEST)


def _gelu(x):
    return 0.5 * x * (1.0 + lax.erf(x * 0.7071067811865476))


def _h1(x_ref, m_ref, wx_ref, wme_ref, wmo_ref, b1_ref):
    m = m_ref[0]
    mlo = lax.bitcast_convert_type(m << 16, jnp.float32)
    mhi = lax.bitcast_convert_type(m & jnp.int32(-65536), jnp.float32)
    return (jnp.dot(wx_ref[...], x_ref[0], **_DOT)
            + jnp.dot(wme_ref[...], mlo, **_DOT)
            + jnp.dot(wmo_ref[...], mhi, **_DOT) + b1_ref[...])


def _norm_gelu(h, s_ref, g_ref, be_ref):
    inv = 1.0 / NSAMP
    mean = s_ref[:, 0:1] * inv
    var = s_ref[:, 1:2] * inv - mean * mean
    scale = g_ref[...] * lax.rsqrt(var + 1e-5)
    return _gelu((h - mean) * scale + be_ref[...])


def _stats(h, s_ref, is_first):
    @pl.when(is_first)
    def _():
        s_ref[...] = jnp.zeros_like(s_ref)

    s_ref[...] += jnp.concatenate(
        [jnp.sum(h, 1, keepdims=True), jnp.sum(h * h, 1, keepdims=True)], 1)


def _mlp(x_ref, m_ref, wx_ref, wme_ref, wmo_ref, b1_ref, g1_ref, be1_ref,
         w2_ref, b2_ref, g2_ref, be2_ref, o_ref, s1_ref, s2_ref, h2s_ref):
    p = pl.program_id(0)
    b = pl.program_id(1)

    @pl.when(p == 0)
    def _():
        _stats(_h1(x_ref, m_ref, wx_ref, wme_ref, wmo_ref, b1_ref),
               s1_ref, b == 0)

    @pl.when(p == 1)
    def _():
        h = _h1(x_ref, m_ref, wx_ref, wme_ref, wmo_ref, b1_ref)
        g = _norm_gelu(h, s1_ref, g1_ref, be1_ref)
        h2 = jnp.dot(w2_ref[...], g, **_DOT) + b2_ref[...]
        h2s_ref[b] = h2
        _stats(h2, s2_ref, b == 0)

    @pl.when(p == 2)
    def _():
        o_ref[0] = _norm_gelu(h2s_ref[b], s2_ref, g2_ref, be2_ref)


def _full(shape):
    return pl.BlockSpec(shape, lambda b: tuple(0 for _ in shape))


def kernel(x, edge_idx, conv1_w, conv1_b, bn1_g, bn1_b,
           conv2_w, conv2_b, bn2_g, bn2_b):
    xc = x.reshape(B, C, N)
    e0 = edge_idx[0].reshape(B, N * K)
    e1 = edge_idx[1].reshape(B, N * K)

    msg = _sc_msg()(xc, e0, e1)

    w1x = conv1_w[:, 0::2]
    w1m = conv1_w[:, 1::2]
    w1me = w1m[:, 0::2]
    w1mo = w1m[:, 1::2]
    b1 = conv1_b.reshape(C2, 1)
    g1 = bn1_g.reshape(C2, 1)
    be1 = bn1_b.reshape(C2, 1)
    b2 = conv2_b.reshape(COUT, 1)
    g2 = bn2_g.reshape(COUT, 1)
    be2 = bn2_b.reshape(COUT, 1)

    xspec = pl.BlockSpec((1, C, N), lambda p, b: (b, 0, 0))
    mspec = pl.BlockSpec((1, CP, N), lambda p, b: (b, 0, 0))

    def _fullpb(shape):
        return pl.BlockSpec(shape, lambda p, b: tuple(0 for _ in shape))

    out = pl.pallas_call(
        _mlp,
        grid=(3, B),
        in_specs=[xspec, mspec, _fullpb((C2, C)), _fullpb((C2, CP)),
                  _fullpb((C2, CP)), _fullpb((C2, 1)), _fullpb((C2, 1)),
                  _fullpb((C2, 1)), _fullpb((COUT, C2)), _fullpb((COUT, 1)),
                  _fullpb((COUT, 1)), _fullpb((COUT, 1))],
        out_specs=pl.BlockSpec((1, COUT, N), lambda p, b: (b, 0, 0)),
        out_shape=jax.ShapeDtypeStruct((B, COUT, N), jnp.float32),
        scratch_shapes=[
            pltpu.VMEM((C2, 2), jnp.float32),
            pltpu.VMEM((COUT, 2), jnp.float32),
            pltpu.VMEM((B, COUT, N), jnp.float32),
        ],
    )(xc, msg, w1x, w1me, w1mo, b1, g1, be1, conv2_w, b2, g2, be2)

    return out.reshape(B, COUT, 32, 32)


# speedup vs baseline: 1.4354x; 1.0248x over previous
import functools

import jax
import jax.numpy as jnp
from jax import lax
from jax.experimental import pallas as pl
from jax.experimental.pallas import tpu as pltpu
from jax.experimental.pallas import tpu_sc as plsc

B, C, N, K = 16, 96, 1024, 16
C2 = 2 * C
COUT = 96
NSAMP = B * N
GN = 128
NSUB = GN // 16


NODES_PER_W = (B * N) // 32
NGROUPS = NODES_PER_W // GN
KH = K // 2
CP = C // 2


CCH = 32


def _sc_body(xc_hbm, e_hbm, msg_hbm, chunk, table,
             idx0a, idx1a, idx0b, idx1b, msgbufa, msgbufb,
             stab, sina, sinb, souta, soutb):
    cid = lax.axis_index("c")
    sid = lax.axis_index("s")
    wid = sid * 2 + cid
    b = wid // 2
    half = wid % 2
    lanes = lax.iota(jnp.int32, 16)
    ninf = jnp.full((32,), -jnp.inf, jnp.bfloat16)

    idxs = [(idx0a, idx1a), (idx0b, idx1b)]
    mbufs = [msgbufa, msgbufb]
    sins = [sina, sinb]
    souts = [souta, soutb]

    def idx_start(g, slot):
        n0 = half * NODES_PER_W + g * GN
        i0, i1 = idxs[slot]
        h0 = pltpu.async_copy(e_hbm.at[0, b, pl.ds(n0 * K, GN * K)], i0,
                              sins[slot])
        h1 = pltpu.async_copy(e_hbm.at[1, b, pl.ds(n0 * K, GN * K)], i1,
                              sins[slot])
        return (h0, h1)

    pend = idx_start(0, 0)

    for ci in range(C // CCH):
        pltpu.async_copy(xc_hbm.at[b, pl.ds(ci * CCH, CCH)], chunk,
                         stab).wait()

        def packv(v, pc_):
            for pp in range(CCH // 2):
                a = chunk[2 * pp, pl.ds(v * 16, 16)]
                o = chunk[2 * pp + 1, pl.ds(v * 16, 16)]
                w = plsc.bitcast(
                    plsc.pack(a, o, format=plsc.PackFormat.INTERLEAVED),
                    jnp.int32)
                table[ci * (CCH // 2) + pp, pl.ds(v * 16, 16)] = w
            return pc_

        lax.fori_loop(0, N // 16, packv, 0)

    out_handles = [None, None]
    for g in range(NGROUPS):
        slot = g % 2
        for h in pend:
            h.wait()
        if g + 1 < NGROUPS:
            pend = idx_start(g + 1, (g + 1) % 2)
        if out_handles[slot] is not None:
            out_handles[slot].wait()
        idx0, idx1 = idxs[slot]
        msgbuf = mbufs[slot]

        def sub(sg, sc_):
            nodes = sg * 16 + lanes

            def khalf(k0, first):
                rs = [plsc.load_gather(idx0, [nodes * K + (k0 + k)])
                      for k in range(KH)]
                rd = [plsc.load_gather(idx1, [nodes * K + (k0 + k)])
                      for k in range(KH)]

                def cbody(c, cc):
                    cvec = jnp.full((16,), c, jnp.int32)
                    acc = ninf
                    for k in range(KH):
                        s = plsc.load_gather(table, [cvec, rs[k]])
                        d = plsc.load_gather(table, [cvec, rd[k]])
                        diff = (plsc.bitcast(s, jnp.bfloat16)
                                - plsc.bitcast(d, jnp.bfloat16))
                        acc = jnp.maximum(acc, diff)
                    if not first:
                        prev = plsc.load_gather(msgbuf, [cvec, nodes])
                        acc = jnp.maximum(acc, plsc.bitcast(prev, jnp.bfloat16))
                    plsc.store_scatter(msgbuf, [cvec, nodes],
                                       plsc.bitcast(acc, jnp.int32))
                    return cc

                lax.fori_loop(0, CP, cbody, 0)

            khalf(0, True)
            khalf(KH, False)
            return sc_

        lax.fori_loop(0, NSUB, sub, 0)
        n0 = half * NODES_PER_W + g * GN
        out_handles[slot] = pltpu.async_copy(
            msgbuf, msg_hbm.at[b, :, pl.ds(n0, GN)], souts[slot])

    for h in out_handles:
        if h is not None:
            h.wait()


@functools.cache
def _sc_msg():
    return pl.kernel(
        _sc_body,
        out_type=jax.ShapeDtypeStruct((B, CP, N), jnp.int32),
        mesh=plsc.VectorSubcoreMesh(core_axis_name="c", subcore_axis_name="s",
                                    num_cores=2, num_subcores=16),
        compiler_params=pltpu.CompilerParams(needs_layout_passes=False),
        scratch_types=[
            pltpu.VMEM((CCH, N), jnp.float32),
            pltpu.VMEM((CP, N), jnp.int32),
            pltpu.VMEM((GN * K,), jnp.int32),
            pltpu.VMEM((GN * K,), jnp.int32),
            pltpu.VMEM((GN * K,), jnp.int32),
            pltpu.VMEM((GN * K,), jnp.int32),
            pltpu.VMEM((CP, GN), jnp.int32),
            pltpu.VMEM((CP, GN), jnp.int32),
            pltpu.SemaphoreType.DMA,
            pltpu.SemaphoreType.DMA,
            pltpu.SemaphoreType.DMA,
            pltpu.SemaphoreType.DMA,
            pltpu.SemaphoreType.DMA,
        ],
    )


_DOT = dict(preferred_element_type=jnp.float32, precision=lax.Precision.HIGHEST)


def _gelu(x):
    return 0.5 * x * (1.0 + lax.erf(x * 0.7071067811865476))


def _h1(x_ref, m_ref, wx_ref, wme_ref, wmo_ref, b1_ref):
    m = m_ref[0]
    mlo = lax.bitcast_convert_type(m << 16, jnp.float32)
    mhi = lax.bitcast_convert_type(m & jnp.int32(-65536), jnp.float32)
    return (jnp.dot(wx_ref[...], x_ref[0], **_DOT)
            + jnp.dot(wme_ref[...], mlo, **_DOT)
            + jnp.dot(wmo_ref[...], mhi, **_DOT) + b1_ref[...])


def _norm_gelu(h, s_ref, g_ref, be_ref):
    inv = 1.0 / NSAMP
    mean = s_ref[:, 0:1] * inv
    var = s_ref[:, 1:2] * inv - mean * mean
    scale = g_ref[...] * lax.rsqrt(var + 1e-5)
    return _gelu((h - mean) * scale + be_ref[...])


def _stats(h, s_ref, is_first):
    @pl.when(is_first)
    def _():
        s_ref[...] = jnp.zeros_like(s_ref)

    s_ref[...] += jnp.concatenate(
        [jnp.sum(h, 1, keepdims=True), jnp.sum(h * h, 1, keepdims=True)], 1)


def _mlp(x_ref, m_ref, wx_ref, wme_ref, wmo_ref, b1_ref, g1_ref, be1_ref,
         w2_ref, b2_ref, g2_ref, be2_ref, o_ref, s1_ref, s2_ref, h2s_ref):
    p = pl.program_id(0)
    b = pl.program_id(1)

    @pl.when(p == 0)
    def _():
        _stats(_h1(x_ref, m_ref, wx_ref, wme_ref, wmo_ref, b1_ref),
               s1_ref, b == 0)

    @pl.when(p == 1)
    def _():
        h = _h1(x_ref, m_ref, wx_ref, wme_ref, wmo_ref, b1_ref)
        g = _norm_gelu(h, s1_ref, g1_ref, be1_ref)
        h2 = jnp.dot(w2_ref[...], g, **_DOT) + b2_ref[...]
        h2s_ref[b] = h2
        _stats(h2, s2_ref, b == 0)

    @pl.when(p == 2)
    def _():
        o_ref[0] = _norm_gelu(h2s_ref[b], s2_ref, g2_ref, be2_ref)


def _full(shape):
    return pl.BlockSpec(shape, lambda b: tuple(0 for _ in shape))


def kernel(x, edge_idx, conv1_w, conv1_b, bn1_g, bn1_b,
           conv2_w, conv2_b, bn2_g, bn2_b):
    xc = x.reshape(B, C, N)
    ef = edge_idx.reshape(2, B, N * K)

    msg = _sc_msg()(xc, ef)

    w1x = conv1_w[:, 0::2]
    w1m = conv1_w[:, 1::2]
    w1me = w1m[:, 0::2]
    w1mo = w1m[:, 1::2]
    b1 = conv1_b.reshape(C2, 1)
    g1 = bn1_g.reshape(C2, 1)
    be1 = bn1_b.reshape(C2, 1)
    b2 = conv2_b.reshape(COUT, 1)
    g2 = bn2_g.reshape(COUT, 1)
    be2 = bn2_b.reshape(COUT, 1)

    xspec = pl.BlockSpec((1, C, N), lambda p, b: (b, 0, 0))
    mspec = pl.BlockSpec((1, CP, N), lambda p, b: (b, 0, 0))

    def _fullpb(shape):
        return pl.BlockSpec(shape, lambda p, b: tuple(0 for _ in shape))

    out = pl.pallas_call(
        _mlp,
        grid=(3, B),
        in_specs=[xspec, mspec, _fullpb((C2, C)), _fullpb((C2, CP)),
                  _fullpb((C2, CP)), _fullpb((C2, 1)), _fullpb((C2, 1)),
                  _fullpb((C2, 1)), _fullpb((COUT, C2)), _fullpb((COUT, 1)),
                  _fullpb((COUT, 1)), _fullpb((COUT, 1))],
        out_specs=pl.BlockSpec((1, COUT, N), lambda p, b: (b, 0, 0)),
        out_shape=jax.ShapeDtypeStruct((B, COUT, N), jnp.float32),
        scratch_shapes=[
            pltpu.VMEM((C2, 2), jnp.float32),
            pltpu.VMEM((COUT, 2), jnp.float32),
            pltpu.VMEM((B, COUT, N), jnp.float32),
        ],
    )(xc, msg, w1x, w1me, w1mo, b1, g1, be1, conv2_w, b2, g2, be2)

    return out.reshape(B, COUT, 32, 32)


# speedup vs baseline: 1.4373x; 1.0013x over previous
import functools

import jax
import jax.numpy as jnp
from jax import lax
from jax.experimental import pallas as pl
from jax.experimental.pallas import tpu as pltpu
from jax.experimental.pallas import tpu_sc as plsc

B, C, N, K = 16, 96, 1024, 16
C2 = 2 * C
COUT = 96
NSAMP = B * N
GN = 128
NSUB = GN // 16


NODES_PER_W = (B * N) // 32
NGROUPS = NODES_PER_W // GN
KH = K // 2
CP = C // 2


CCH = 32


def _sc_body(xc_hbm, e_hbm, msg_hbm, chunk, table,
             idx0a, idx1a, idx0b, idx1b, msgbufa, msgbufb,
             stab, sina, sinb, souta, soutb):
    cid = lax.axis_index("c")
    sid = lax.axis_index("s")
    wid = sid * 2 + cid
    b = wid // 2
    half = wid % 2
    lanes = lax.iota(jnp.int32, 16)
    ninf = jnp.full((32,), -jnp.inf, jnp.bfloat16)

    idxs = [(idx0a, idx1a), (idx0b, idx1b)]
    mbufs = [msgbufa, msgbufb]
    sins = [sina, sinb]
    souts = [souta, soutb]

    def idx_start(g, slot):
        n0 = half * NODES_PER_W + g * GN
        i0, i1 = idxs[slot]
        h0 = pltpu.async_copy(e_hbm.at[0, b, pl.ds(n0 * K, GN * K)], i0,
                              sins[slot])
        h1 = pltpu.async_copy(e_hbm.at[1, b, pl.ds(n0 * K, GN * K)], i1,
                              sins[slot])
        return (h0, h1)

    pend = idx_start(0, 0)

    for ci in range(C // CCH):
        pltpu.async_copy(xc_hbm.at[b, pl.ds(ci * CCH, CCH)], chunk,
                         stab).wait()

        def packv(v, pc_):
            for pp in range(CCH // 2):
                a = chunk[2 * pp, pl.ds(v * 16, 16)]
                o = chunk[2 * pp + 1, pl.ds(v * 16, 16)]
                w = plsc.bitcast(
                    plsc.pack(a, o, format=plsc.PackFormat.INTERLEAVED),
                    jnp.int32)
                table[ci * (CCH // 2) + pp, pl.ds(v * 16, 16)] = w
            return pc_

        lax.fori_loop(0, N // 16, packv, 0)

    out_handles = [None, None]
    for g in range(NGROUPS):
        slot = g % 2
        for h in pend:
            h.wait()
        if g + 1 < NGROUPS:
            pend = idx_start(g + 1, (g + 1) % 2)
        if out_handles[slot] is not None:
            out_handles[slot].wait()
        idx0, idx1 = idxs[slot]
        msgbuf = mbufs[slot]

        def sub(sg, sc_):
            nodes = sg * 16 + lanes

            def khalf(k0, first):
                rs = [plsc.load_gather(idx0, [nodes * K + (k0 + k)])
                      for k in range(KH)]
                rd = [plsc.load_gather(idx1, [nodes * K + (k0 + k)])
                      for k in range(KH)]

                def cbody(c, cc):
                    cvec = jnp.full((16,), c, jnp.int32)
                    acc = ninf
                    for k in range(KH):
                        s = plsc.load_gather(table, [cvec, rs[k]])
                        d = plsc.load_gather(table, [cvec, rd[k]])
                        diff = (plsc.bitcast(s, jnp.bfloat16)
                                - plsc.bitcast(d, jnp.bfloat16))
                        acc = jnp.maximum(acc, diff)
                    if not first:
                        prev = plsc.load_gather(msgbuf, [cvec, nodes])
                        acc = jnp.maximum(acc, plsc.bitcast(prev, jnp.bfloat16))
                    plsc.store_scatter(msgbuf, [cvec, nodes],
                                       plsc.bitcast(acc, jnp.int32))
                    return cc

                lax.fori_loop(0, CP, cbody, 0)

            khalf(0, True)
            khalf(KH, False)
            return sc_

        lax.fori_loop(0, NSUB, sub, 0)
        n0 = half * NODES_PER_W + g * GN
        out_handles[slot] = pltpu.async_copy(
            msgbuf, msg_hbm.at[b, :, pl.ds(n0, GN)], souts[slot])

    for h in out_handles:
        if h is not None:
            h.wait()


@functools.cache
def _sc_msg():
    return pl.kernel(
        _sc_body,
        out_type=jax.ShapeDtypeStruct((B, CP, N), jnp.int32),
        mesh=plsc.VectorSubcoreMesh(core_axis_name="c", subcore_axis_name="s",
                                    num_cores=2, num_subcores=16),
        compiler_params=pltpu.CompilerParams(needs_layout_passes=False),
        scratch_types=[
            pltpu.VMEM((CCH, N), jnp.float32),
            pltpu.VMEM((CP, N), jnp.int32),
            pltpu.VMEM((GN * K,), jnp.int32),
            pltpu.VMEM((GN * K,), jnp.int32),
            pltpu.VMEM((GN * K,), jnp.int32),
            pltpu.VMEM((GN * K,), jnp.int32),
            pltpu.VMEM((CP, GN), jnp.int32),
            pltpu.VMEM((CP, GN), jnp.int32),
            pltpu.SemaphoreType.DMA,
            pltpu.SemaphoreType.DMA,
            pltpu.SemaphoreType.DMA,
            pltpu.SemaphoreType.DMA,
            pltpu.SemaphoreType.DMA,
        ],
    )


_DOT = dict(preferred_element_type=jnp.float32, precision=lax.Precision.HIGHEST)


def _gelu(x):
    return 0.5 * x * (1.0 + lax.erf(x * 0.7071067811865476))


def _h1(x_ref, m_ref, wx_ref, wme_ref, wmo_ref, b1_ref):
    m = m_ref[0]
    mlo = lax.bitcast_convert_type(m << 16, jnp.float32)
    mhi = lax.bitcast_convert_type(m & jnp.int32(-65536), jnp.float32)
    return (jnp.dot(wx_ref[...], x_ref[0], **_DOT)
            + jnp.dot(wme_ref[...], mlo, **_DOT)
            + jnp.dot(wmo_ref[...], mhi, **_DOT) + b1_ref[...])


def _norm_gelu(h, s_ref, g_ref, be_ref):
    inv = 1.0 / NSAMP
    mean = s_ref[:, 0:1] * inv
    var = s_ref[:, 1:2] * inv - mean * mean
    scale = g_ref[...] * lax.rsqrt(var + 1e-5)
    return _gelu((h - mean) * scale + be_ref[...])


def _stats(h, s_ref, is_first):
    @pl.when(is_first)
    def _():
        s_ref[...] = jnp.zeros_like(s_ref)

    s_ref[...] += jnp.concatenate(
        [jnp.sum(h, 1, keepdims=True), jnp.sum(h * h, 1, keepdims=True)], 1)


def _mlp(x_ref, m_ref, wx_ref, wme_ref, wmo_ref, b1_ref, g1_ref, be1_ref,
         w2_ref, b2_ref, g2_ref, be2_ref, o_ref, s1_ref, s2_ref, h2s_ref):
    p = pl.program_id(0)
    b = pl.program_id(1)

    @pl.when(p == 0)
    def _():
        _stats(_h1(x_ref, m_ref, wx_ref, wme_ref, wmo_ref, b1_ref),
               s1_ref, b == 0)

    @pl.when(p == 1)
    def _():
        h = _h1(x_ref, m_ref, wx_ref, wme_ref, wmo_ref, b1_ref)
        g = _norm_gelu(h, s1_ref, g1_ref, be1_ref)
        h2 = jnp.dot(w2_ref[...], g, **_DOT) + b2_ref[...]
        h2s_ref[b] = h2
        _stats(h2, s2_ref, b == 0)

    @pl.when(p == 2)
    def _():
        o_ref[0] = _norm_gelu(h2s_ref[b], s2_ref, g2_ref, be2_ref)


def kernel(x, edge_idx, conv1_w, conv1_b, bn1_g, bn1_b,
           conv2_w, conv2_b, bn2_g, bn2_b):
    xc = x.reshape(B, C, N)
    ef = edge_idx.reshape(2, B, N * K)

    msg = _sc_msg()(xc, ef)

    w1x = conv1_w[:, 0::2]
    w1m = conv1_w[:, 1::2]
    w1me = w1m[:, 0::2]
    w1mo = w1m[:, 1::2]
    b1 = conv1_b.reshape(C2, 1)
    g1 = bn1_g.reshape(C2, 1)
    be1 = bn1_b.reshape(C2, 1)
    b2 = conv2_b.reshape(COUT, 1)
    g2 = bn2_g.reshape(COUT, 1)
    be2 = bn2_b.reshape(COUT, 1)

    xspec = pl.BlockSpec((1, C, N), lambda p, b: (b, 0, 0))
    mspec = pl.BlockSpec((1, CP, N), lambda p, b: (b, 0, 0))

    def _fullpb(shape):
        return pl.BlockSpec(shape, lambda p, b: tuple(0 for _ in shape))

    out = pl.pallas_call(
        _mlp,
        grid=(3, B),
        in_specs=[xspec, mspec, _fullpb((C2, C)), _fullpb((C2, CP)),
                  _fullpb((C2, CP)), _fullpb((C2, 1)), _fullpb((C2, 1)),
                  _fullpb((C2, 1)), _fullpb((COUT, C2)), _fullpb((COUT, 1)),
                  _fullpb((COUT, 1)), _fullpb((COUT, 1))],
        out_specs=pl.BlockSpec((1, COUT, N), lambda p, b: (b, 0, 0)),
        out_shape=jax.ShapeDtypeStruct((B, COUT, N), jnp.float32),
        scratch_shapes=[
            pltpu.VMEM((C2, 2), jnp.float32),
            pltpu.VMEM((COUT, 2), jnp.float32),
            pltpu.VMEM((B, COUT, N), jnp.float32),
        ],
    )(xc, msg, w1x, w1me, w1mo, b1, g1, be1, conv2_w, b2, g2, be2)

    return out.reshape(B, COUT, 32, 32)
